# baseline (speedup 1.0000x reference)
import functools

import jax
import jax.numpy as jnp
from jax.experimental import pallas as pl
from jax.experimental.pallas import tpu as pltpu
from jax.experimental.pallas import tpu_sc as plsc

D = 512
H = 8
DK = 64
CHUNK = 64
E = 4
TOPK = 2
MULT = 4
CAPF = 1.25
LANES = 128

BT = 512
BTF = 256
NW = 32
CHC = 32


def _attn_router_body(x_ref, n1_ref, n2_ref, wcat_ref, wo_ref, bcat_ref,
                      ob_ref, cos_ref, sin_ref, rw_ref, rb_ref,
                      x2_ref, xn2_ref, spos1_ref, spos2_ref, g1_ref, g2_ref,
                      stats_ref,
                      S_ref, Z_ref, cnt_ref, *, cap, nt):
    b = pl.program_id(0)
    t = pl.program_id(1)

    @pl.when(t == 0)
    def _():
        S_ref[...] = jnp.zeros_like(S_ref)
        Z_ref[...] = jnp.zeros_like(Z_ref)

    @pl.when(jnp.logical_and(b == 0, t == 0))
    def _():
        cnt_ref[...] = jnp.zeros_like(cnt_ref)

    xb = x_ref[0]
    ms = jnp.mean(xb * xb, axis=-1, keepdims=True)
    xn = xb * jax.lax.rsqrt(ms) * n1_ref[...]

    y6 = xn @ wcat_ref[...] + bcat_ref[...]
    q = y6[:, 0:D]
    qp = y6[:, D:2 * D]
    k = y6[:, 2 * D:3 * D]
    kp = y6[:, 3 * D:4 * D]
    v = y6[:, 4 * D:5 * D]
    g = 1.0 / (1.0 + jnp.exp(-y6[:, 5 * D:6 * D]))

    cosb = cos_ref[...]
    sinb = sin_ref[...]
    qr = q * cosb + qp * sinb
    kr = k * cosb + kp * sinb
    qe = jnp.where(qr > 0, qr + 1.0, jnp.exp(qr))
    ke = jnp.where(kr > 0, kr + 1.0, jnp.exp(kr))

    ri = jax.lax.broadcasted_iota(jnp.int32, (D, D), 0) // DK
    ci = jax.lax.broadcasted_iota(jnp.int32, (D, D), 1) // DK
    bdf = (ri == ci).astype(jnp.float32)

    outs = []
    for c in range(BT // CHUNK):
        sl = slice(c * CHUNK, (c + 1) * CHUNK)
        qc, kc, vc, gc = qe[sl], ke[sl], v[sl], g[sl]
        m = jax.lax.dot_general(kc, vc, (((0,), (0,)), ((), ())),
                                preferred_element_type=jnp.float32)
        S_ref[...] += m * bdf
        Z_ref[0:1, :] += jnp.sum(kc, axis=0, keepdims=True)
        num = jax.lax.dot_general(qc, S_ref[...], (((1,), (0,)), ((), ())),
                                  preferred_element_type=jnp.float32)
        den = jax.lax.dot_general(qc * Z_ref[0:1, :], bdf,
                                  (((1,), (0,)), ((), ())),
                                  preferred_element_type=jnp.float32) + 1e-6
        outs.append(gc * num / den)
    attn = jnp.concatenate(outs, axis=0)
    x2 = xb + attn @ wo_ref[...] + ob_ref[...]
    x2_ref[0] = x2

    ms2 = jnp.mean(x2 * x2, axis=-1, keepdims=True)
    xn2 = x2 * jax.lax.rsqrt(ms2) * n2_ref[...]
    xn2_ref[...] = xn2

    logits = xn2 @ rw_ref[...] + rb_ref[...]
    mx = jnp.max(logits, axis=-1, keepdims=True)
    pe = jnp.exp(logits - mx)
    probs = pe / jnp.sum(pe, axis=-1, keepdims=True)
    cnt_ref[2:3, :] += jnp.sum(probs, axis=0, keepdims=True)

    lane = jax.lax.broadcasted_iota(jnp.int32, (BT, LANES), 1)
    m1 = jnp.max(probs, axis=-1, keepdims=True)
    i1 = jnp.min(jnp.where(probs == m1, lane, LANES), axis=-1, keepdims=True)
    pwo = jnp.where(lane == i1, -1.0, probs)
    m2 = jnp.max(pwo, axis=-1, keepdims=True)
    i2 = jnp.min(jnp.where(pwo == m2, lane, LANES), axis=-1, keepdims=True)
    tsum = m1 + m2
    tp1 = m1 / tsum
    tp2 = m2 / tsum

    oh1 = (lane == i1).astype(jnp.float32)
    oh2 = (lane == i2).astype(jnp.float32)
    mm = oh1 + oh2
    lt = (jax.lax.broadcasted_iota(jnp.int32, (BT, BT), 1) <
          jax.lax.broadcasted_iota(jnp.int32, (BT, BT), 0)).astype(jnp.float32)
    excl = jax.lax.dot_general(lt, mm, (((1,), (0,)), ((), ())),
                               preferred_element_type=jnp.float32)
    rank = cnt_ref[0:1, :] + excl
    keepm = (rank < float(cap)).astype(jnp.float32)
    keep1 = oh1 * keepm
    keep2 = oh2 * keepm
    cnt_ref[1:2, :] += jnp.sum(keep1 + keep2, axis=0, keepdims=True)
    cnt_ref[0:1, :] += jnp.sum(mm, axis=0, keepdims=True)
    stats_ref[...] = cnt_ref[...]

    nslot = E * cap
    rank1 = jnp.sum(rank * oh1, axis=-1, keepdims=True).astype(jnp.int32)
    rank2 = jnp.sum(rank * oh2, axis=-1, keepdims=True).astype(jnp.int32)
    k1 = jnp.sum(keep1, axis=-1, keepdims=True)
    k2 = jnp.sum(keep2, axis=-1, keepdims=True)
    ntok = (jax.lax.broadcasted_iota(jnp.int32, (BT, 1), 0)
            + (b * nt + t) * BT)
    trash = nslot + jnp.mod(ntok, BTF)
    spos1_ref[...] = jnp.where(k1 > 0, i1 * cap + rank1, trash)
    spos2_ref[...] = jnp.where(k2 > 0, i2 * cap + rank2, trash)
    g1_ref[...] = jnp.broadcast_to(k1 * tp1, (BT, 16))
    g2_ref[...] = jnp.broadcast_to(k2 * tp2, (BT, 16))


def _ffn_body(xe_ref, w1_ref, b1_ref, w2_ref, b2_ref, y_ref, *, nsteps):
    s = pl.program_id(0)

    @pl.when(s < nsteps)
    def _():
        xv = xe_ref[...]
        h = xv @ w1_ref[0] + b1_ref[0]
        h = 0.5 * h * (1.0 + jax.lax.erf(h * 0.7071067811865476))
        y_ref[...] = h @ w2_ref[0] + b2_ref[0]

    @pl.when(s == nsteps)
    def _():
        y_ref[...] = jnp.zeros_like(y_ref)


def _dispatch_body(xn2_hbm, s1_hbm, s2_hbm, xe_hbm,
                   idx1_v, idx2_v, rows_v, sem1, sem2, *, cht):
    wid = jax.lax.axis_index("s") * 2 + jax.lax.axis_index("c")
    base = wid * cht
    pltpu.sync_copy(s1_hbm.at[pl.ds(base, cht)], idx1_v)
    pltpu.sync_copy(s2_hbm.at[pl.ds(base, cht)], idx2_v)
    pltpu.sync_copy(xn2_hbm.at[pl.ds(base, cht)], rows_v)
    c1 = pltpu.async_copy(rows_v, xe_hbm.at[idx1_v], sem1)
    c2 = pltpu.async_copy(rows_v, xe_hbm.at[idx2_v], sem2)
    c1.wait()
    c2.wait()


def _combine_body(x2_hbm, y_hbm, s1_hbm, s2_hbm, g1_hbm, g2_hbm, out_hbm,
                  i1a, i2a, i1b, i2b, g1_v, g2_v,
                  y1a, y2a, y1b, y2b, acc_v,
                  s1a, s2a, s1b, s2b, *, cht):
    wid = jax.lax.axis_index("s") * 2 + jax.lax.axis_index("c")
    nch = cht // CHC
    ibufs = [(i1a, i2a), (i1b, i2b)]
    ybufs = [(y1a, y2a), (y1b, y2b)]
    sems = [(s1a, s2a), (s1b, s2b)]
    copies = [None, None]

    def start(c):
        bi = c % 2
        i1, i2 = ibufs[bi]
        y1, y2 = ybufs[bi]
        sa, sb = sems[bi]
        b = wid * cht + c * CHC
        pltpu.sync_copy(s1_hbm.at[pl.ds(b, CHC)], i1)
        pltpu.sync_copy(s2_hbm.at[pl.ds(b, CHC)], i2)
        copies[bi] = (pltpu.async_copy(y_hbm.at[i1], y1, sa),
                      pltpu.async_copy(y_hbm.at[i2], y2, sb))

    start(0)
    for c in range(nch):
        bi = c % 2
        if c + 1 < nch:
            start(c + 1)
        b = wid * cht + c * CHC
        pltpu.sync_copy(x2_hbm.at[pl.ds(b, CHC)], acc_v)
        pltpu.sync_copy(g1_hbm.at[pl.ds(b, CHC)], g1_v)
        pltpu.sync_copy(g2_hbm.at[pl.ds(b, CHC)], g2_v)
        cy1, cy2 = copies[bi]
        cy1.wait()
        cy2.wait()
        y1, y2 = ybufs[bi]

        def row_body(r, carry, y1=y1, y2=y2):
            gv1 = g1_v[r, :]
            gv2 = g2_v[r, :]
            for j in range(D // 16):
                sl = pl.ds(j * 16, 16)
                acc_v[r, sl] = (acc_v[r, sl] + gv1 * y1[r, sl]
                                + gv2 * y2[r, sl])
            return carry

        jax.lax.fori_loop(0, CHC, row_body, 0)
        pltpu.sync_copy(acc_v, out_hbm.at[pl.ds(b, CHC)])


def _swap_halves(w):
    shp = w.shape
    wr = w.reshape(shp[:-1] + (H, 2, DK // 2))
    return jnp.flip(wr, axis=-2).reshape(shp)


def kernel(x, norm1_scale, norm2_scale, qw, qb, kw, kb, vw, vb, gw, gb, ow, ob,
           router_w, router_b, e_w1, e_b1, e_w2, e_b2):
    B, T, _ = x.shape
    N = B * T
    NT = T // BT
    cap = int(CAPF * (N / E))
    nslot = E * cap
    tbl = nslot + BTF
    hidden = D * MULT
    cht = N // NW

    qwp = _swap_halves(qw)
    kwp = _swap_halves(kw)
    qbp = _swap_halves(qb)
    kbp = _swap_halves(kb)
    wcat = jnp.concatenate([qw, qwp, kw, kwp, vw, gw], axis=1)
    bcat = jnp.concatenate([qb, qbp, kb, kbp, vb, gb]).reshape(1, 6 * D)

    half = DK // 2
    freqs = 1.0 / (10000.0 ** (jnp.arange(half, dtype=jnp.float32) / half))
    f = jnp.outer(jnp.arange(T, dtype=jnp.float32), freqs)
    cosb = jnp.cos(f)
    sinb = jnp.sin(f)
    cosF = jnp.tile(jnp.concatenate([cosb, cosb], axis=1), (1, H))
    sinF = jnp.tile(jnp.concatenate([-sinb, sinb], axis=1), (1, H))

    rw_pad = jnp.zeros((D, LANES), jnp.float32).at[:, :E].set(router_w)
    rb_pad = jnp.full((1, LANES), -1e30, jnp.float32).at[0, :E].set(router_b)

    body = functools.partial(_attn_router_body, cap=cap, nt=NT)
    x2, xn2, spos1, spos2, g1, g2, stats = pl.pallas_call(
        body,
        grid=(B, NT),
        in_specs=[
            pl.BlockSpec((1, BT, D), lambda b, t: (b, t, 0)),
            pl.BlockSpec((1, D), lambda b, t: (0, 0)),
            pl.BlockSpec((1, D), lambda b, t: (0, 0)),
            pl.BlockSpec((D, 6 * D), lambda b, t: (0, 0)),
            pl.BlockSpec((D, D), lambda b, t: (0, 0)),
            pl.BlockSpec((1, 6 * D), lambda b, t: (0, 0)),
            pl.BlockSpec((1, D), lambda b, t: (0, 0)),
            pl.BlockSpec((BT, D), lambda b, t: (t, 0)),
            pl.BlockSpec((BT, D), lambda b, t: (t, 0)),
            pl.BlockSpec((D, LANES), lambda b, t: (0, 0)),
            pl.BlockSpec((1, LANES), lambda b, t: (0, 0)),
        ],
        out_specs=[
            pl.BlockSpec((1, BT, D), lambda b, t: (b, t, 0)),
            pl.BlockSpec((BT, D), lambda b, t: (b * (T // BT) + t, 0)),
            pl.BlockSpec((BT, 1), lambda b, t: (b * (T // BT) + t, 0)),
            pl.BlockSpec((BT, 1), lambda b, t: (b * (T // BT) + t, 0)),
            pl.BlockSpec((BT, 16), lambda b, t: (b * (T // BT) + t, 0)),
            pl.BlockSpec((BT, 16), lambda b, t: (b * (T // BT) + t, 0)),
            pl.BlockSpec((8, LANES), lambda b, t: (0, 0)),
        ],
        out_shape=[
            jax.ShapeDtypeStruct((B, T, D), jnp.float32),
            jax.ShapeDtypeStruct((N, D), jnp.float32),
            jax.ShapeDtypeStruct((N, 1), jnp.int32),
            jax.ShapeDtypeStruct((N, 1), jnp.int32),
            jax.ShapeDtypeStruct((N, 16), jnp.float32),
            jax.ShapeDtypeStruct((N, 16), jnp.float32),
            jax.ShapeDtypeStruct((8, LANES), jnp.float32),
        ],
        scratch_shapes=[
            pltpu.VMEM((D, D), jnp.float32),
            pltpu.VMEM((8, D), jnp.float32),
            pltpu.VMEM((8, LANES), jnp.float32),
        ],
        compiler_params=pltpu.CompilerParams(
            dimension_semantics=("arbitrary", "arbitrary")),
    )(x, norm1_scale.reshape(1, D), norm2_scale.reshape(1, D),
      wcat, ow, bcat, ob.reshape(1, D), cosF, sinF, rw_pad, rb_pad)

    s1f = spos1.reshape(N)
    s2f = spos2.reshape(N)

    mesh = plsc.VectorSubcoreMesh(core_axis_name="c", subcore_axis_name="s",
                                  num_cores=2, num_subcores=16)

    dispatch = pl.kernel(
        functools.partial(_dispatch_body, cht=cht),
        out_type=jax.ShapeDtypeStruct((tbl, D), jnp.float32),
        mesh=mesh,
        scratch_types=[
            pltpu.VMEM((cht,), jnp.int32),
            pltpu.VMEM((cht,), jnp.int32),
            pltpu.VMEM((cht, D), jnp.float32),
            pltpu.SemaphoreType.DMA,
            pltpu.SemaphoreType.DMA,
        ],
    )
    xe = dispatch(xn2, s1f, s2f)

    nbf = cap // BTF
    nsteps = E * nbf
    ffn = functools.partial(_ffn_body, nsteps=nsteps)
    y = pl.pallas_call(
        ffn,
        grid=(nsteps + 1,),
        in_specs=[
            pl.BlockSpec((BTF, D),
                         lambda s: (jnp.minimum(s, E * (cap // BTF) - 1), 0)),
            pl.BlockSpec((1, D, hidden),
                         lambda s: (jnp.minimum(s // (cap // BTF), E - 1), 0, 0)),
            pl.BlockSpec((1, 1, hidden),
                         lambda s: (jnp.minimum(s // (cap // BTF), E - 1), 0, 0)),
            pl.BlockSpec((1, hidden, D),
                         lambda s: (jnp.minimum(s // (cap // BTF), E - 1), 0, 0)),
            pl.BlockSpec((1, 1, D),
                         lambda s: (jnp.minimum(s // (cap // BTF), E - 1), 0, 0)),
        ],
        out_specs=pl.BlockSpec((BTF, D), lambda s: (s, 0)),
        out_shape=jax.ShapeDtypeStruct((nslot + BTF, D), jnp.float32),
        compiler_params=pltpu.CompilerParams(
            dimension_semantics=("arbitrary",)),
    )(xe, e_w1, e_b1.reshape(E, 1, hidden), e_w2, e_b2.reshape(E, 1, D))

    combine = pl.kernel(
        functools.partial(_combine_body, cht=cht),
        out_type=jax.ShapeDtypeStruct((N, D), jnp.float32),
        mesh=mesh,
        scratch_types=[
            pltpu.VMEM((CHC,), jnp.int32),
            pltpu.VMEM((CHC,), jnp.int32),
            pltpu.VMEM((CHC,), jnp.int32),
            pltpu.VMEM((CHC,), jnp.int32),
            pltpu.VMEM((CHC, 16), jnp.float32),
            pltpu.VMEM((CHC, 16), jnp.float32),
            pltpu.VMEM((CHC, D), jnp.float32),
            pltpu.VMEM((CHC, D), jnp.float32),
            pltpu.VMEM((CHC, D), jnp.float32),
            pltpu.VMEM((CHC, D), jnp.float32),
            pltpu.VMEM((CHC, D), jnp.float32),
            pltpu.SemaphoreType.DMA,
            pltpu.SemaphoreType.DMA,
            pltpu.SemaphoreType.DMA,
            pltpu.SemaphoreType.DMA,
        ],
    )
    final = combine(x2.reshape(N, D), y, s1f, s2f, g1, g2)

    imp = stats[2, :E]
    load = stats[1, :E]
    aux = jnp.sum((imp / jnp.sum(imp)) * (load / jnp.sum(load))) * float(E * E)
    return final.reshape(B, T, D), aux


# speedup vs baseline: 1.0284x; 1.0284x over previous
import functools

import jax
import jax.numpy as jnp
from jax.experimental import pallas as pl
from jax.experimental.pallas import tpu as pltpu
from jax.experimental.pallas import tpu_sc as plsc

D = 512
H = 8
DK = 64
CHUNK = 64
E = 4
TOPK = 2
MULT = 4
CAPF = 1.25
LANES = 128

BT = 512
BTF = 256
NW = 32
CHC = 64


def _attn_router_body(x_ref, n1_ref, n2_ref, wcat_ref, wo_ref, bcat_ref,
                      ob_ref, cos_ref, sin_ref, rw_ref, rb_ref,
                      x2_ref, xn2_ref, spos1_ref, spos2_ref, g1_ref, g2_ref,
                      stats_ref,
                      S_ref, Z_ref, cnt_ref, *, cap, nt):
    b = pl.program_id(0)
    t = pl.program_id(1)

    @pl.when(t == 0)
    def _():
        S_ref[...] = jnp.zeros_like(S_ref)
        Z_ref[...] = jnp.zeros_like(Z_ref)

    @pl.when(jnp.logical_and(b == 0, t == 0))
    def _():
        cnt_ref[...] = jnp.zeros_like(cnt_ref)

    xb = x_ref[0]
    ms = jnp.mean(xb * xb, axis=-1, keepdims=True)
    xn = xb * jax.lax.rsqrt(ms) * n1_ref[...]

    y6 = xn @ wcat_ref[...] + bcat_ref[...]
    q = y6[:, 0:D]
    qp = y6[:, D:2 * D]
    k = y6[:, 2 * D:3 * D]
    kp = y6[:, 3 * D:4 * D]
    v = y6[:, 4 * D:5 * D]
    g = 1.0 / (1.0 + jnp.exp(-y6[:, 5 * D:6 * D]))

    cosb = cos_ref[...]
    sinb = sin_ref[...]
    qr = q * cosb + qp * sinb
    kr = k * cosb + kp * sinb
    qe = jnp.where(qr > 0, qr + 1.0, jnp.exp(qr))
    ke = jnp.where(kr > 0, kr + 1.0, jnp.exp(kr))

    ri = jax.lax.broadcasted_iota(jnp.int32, (D, D), 0) // DK
    ci = jax.lax.broadcasted_iota(jnp.int32, (D, D), 1) // DK
    bdf = (ri == ci).astype(jnp.float32)

    outs = []
    for c in range(BT // CHUNK):
        sl = slice(c * CHUNK, (c + 1) * CHUNK)
        qc, kc, vc, gc = qe[sl], ke[sl], v[sl], g[sl]
        m = jax.lax.dot_general(kc, vc, (((0,), (0,)), ((), ())),
                                preferred_element_type=jnp.float32)
        S_ref[...] += m * bdf
        Z_ref[0:1, :] += jnp.sum(kc, axis=0, keepdims=True)
        num = jax.lax.dot_general(qc, S_ref[...], (((1,), (0,)), ((), ())),
                                  preferred_element_type=jnp.float32)
        den = jax.lax.dot_general(qc * Z_ref[0:1, :], bdf,
                                  (((1,), (0,)), ((), ())),
                                  preferred_element_type=jnp.float32) + 1e-6
        outs.append(gc * num / den)
    attn = jnp.concatenate(outs, axis=0)
    x2 = xb + attn @ wo_ref[...] + ob_ref[...]
    x2_ref[0] = x2

    ms2 = jnp.mean(x2 * x2, axis=-1, keepdims=True)
    xn2 = x2 * jax.lax.rsqrt(ms2) * n2_ref[...]
    xn2_ref[...] = xn2

    logits = xn2 @ rw_ref[...] + rb_ref[...]
    mx = jnp.max(logits, axis=-1, keepdims=True)
    pe = jnp.exp(logits - mx)
    probs = pe / jnp.sum(pe, axis=-1, keepdims=True)
    cnt_ref[2:3, :] += jnp.sum(probs, axis=0, keepdims=True)

    lane = jax.lax.broadcasted_iota(jnp.int32, (BT, LANES), 1)
    m1 = jnp.max(probs, axis=-1, keepdims=True)
    i1 = jnp.min(jnp.where(probs == m1, lane, LANES), axis=-1, keepdims=True)
    pwo = jnp.where(lane == i1, -1.0, probs)
    m2 = jnp.max(pwo, axis=-1, keepdims=True)
    i2 = jnp.min(jnp.where(pwo == m2, lane, LANES), axis=-1, keepdims=True)
    tsum = m1 + m2
    tp1 = m1 / tsum
    tp2 = m2 / tsum

    oh1 = (lane == i1).astype(jnp.float32)
    oh2 = (lane == i2).astype(jnp.float32)
    mm = oh1 + oh2
    lt = (jax.lax.broadcasted_iota(jnp.int32, (BT, BT), 1) <
          jax.lax.broadcasted_iota(jnp.int32, (BT, BT), 0)).astype(jnp.float32)
    excl = jax.lax.dot_general(lt, mm, (((1,), (0,)), ((), ())),
                               preferred_element_type=jnp.float32)
    rank = cnt_ref[0:1, :] + excl
    keepm = (rank < float(cap)).astype(jnp.float32)
    keep1 = oh1 * keepm
    keep2 = oh2 * keepm
    cnt_ref[1:2, :] += jnp.sum(keep1 + keep2, axis=0, keepdims=True)
    cnt_ref[0:1, :] += jnp.sum(mm, axis=0, keepdims=True)
    stats_ref[...] = cnt_ref[...]

    nslot = E * cap
    rank1 = jnp.sum(rank * oh1, axis=-1, keepdims=True).astype(jnp.int32)
    rank2 = jnp.sum(rank * oh2, axis=-1, keepdims=True).astype(jnp.int32)
    k1 = jnp.sum(keep1, axis=-1, keepdims=True)
    k2 = jnp.sum(keep2, axis=-1, keepdims=True)
    ntok = (jax.lax.broadcasted_iota(jnp.int32, (BT, 1), 0)
            + (b * nt + t) * BT)
    trash = nslot + jnp.mod(ntok, BTF)
    spos1_ref[...] = jnp.where(k1 > 0, i1 * cap + rank1, trash)
    spos2_ref[...] = jnp.where(k2 > 0, i2 * cap + rank2, trash)
    g1_ref[...] = jnp.broadcast_to(k1 * tp1, (BT, 16))
    g2_ref[...] = jnp.broadcast_to(k2 * tp2, (BT, 16))


def _ffn_body(xe_ref, w1_ref, b1_ref, w2_ref, b2_ref, y_ref, *, nsteps):
    s = pl.program_id(0)

    @pl.when(s < nsteps)
    def _():
        xv = xe_ref[...]
        h = xv @ w1_ref[0] + b1_ref[0]
        h = 0.5 * h * (1.0 + jax.lax.erf(h * 0.7071067811865476))
        y_ref[...] = h @ w2_ref[0] + b2_ref[0]

    @pl.when(s == nsteps)
    def _():
        y_ref[...] = jnp.zeros_like(y_ref)


def _dispatch_body(xn2_hbm, s1_hbm, s2_hbm, xe_hbm,
                   idx1_v, idx2_v, rows_v, sem1, sem2, *, cht):
    wid = jax.lax.axis_index("s") * 2 + jax.lax.axis_index("c")
    base = wid * cht
    pltpu.sync_copy(s1_hbm.at[pl.ds(base, cht)], idx1_v)
    pltpu.sync_copy(s2_hbm.at[pl.ds(base, cht)], idx2_v)
    pltpu.sync_copy(xn2_hbm.at[pl.ds(base, cht)], rows_v)
    c1 = pltpu.async_copy(rows_v, xe_hbm.at[idx1_v], sem1)
    c2 = pltpu.async_copy(rows_v, xe_hbm.at[idx2_v], sem2)
    c1.wait()
    c2.wait()


def _combine_body(x2_hbm, y_hbm, s1_hbm, s2_hbm, g1_hbm, g2_hbm, out_hbm,
                  idx1_v, idx2_v, g1_v, g2_v, y1_v, y2_v, acc_v, sy1, sy2,
                  *, cht):
    wid = jax.lax.axis_index("s") * 2 + jax.lax.axis_index("c")
    for c in range(cht // CHC):
        b = wid * cht + c * CHC
        pltpu.sync_copy(s1_hbm.at[pl.ds(b, CHC)], idx1_v)
        pltpu.sync_copy(s2_hbm.at[pl.ds(b, CHC)], idx2_v)
        cy1 = pltpu.async_copy(y_hbm.at[idx1_v], y1_v, sy1)
        cy2 = pltpu.async_copy(y_hbm.at[idx2_v], y2_v, sy2)
        pltpu.sync_copy(x2_hbm.at[pl.ds(b, CHC)], acc_v)
        pltpu.sync_copy(g1_hbm.at[pl.ds(b, CHC)], g1_v)
        pltpu.sync_copy(g2_hbm.at[pl.ds(b, CHC)], g2_v)
        cy1.wait()
        cy2.wait()

        def row_body(r, carry):
            gv1 = g1_v[r, :]
            gv2 = g2_v[r, :]
            for j in range(D // 16):
                sl = pl.ds(j * 16, 16)
                acc_v[r, sl] = (acc_v[r, sl] + gv1 * y1_v[r, sl]
                                + gv2 * y2_v[r, sl])
            return carry

        jax.lax.fori_loop(0, CHC, row_body, 0)
        pltpu.sync_copy(acc_v, out_hbm.at[pl.ds(b, CHC)])


def _swap_halves(w):
    shp = w.shape
    wr = w.reshape(shp[:-1] + (H, 2, DK // 2))
    return jnp.flip(wr, axis=-2).reshape(shp)


def kernel(x, norm1_scale, norm2_scale, qw, qb, kw, kb, vw, vb, gw, gb, ow, ob,
           router_w, router_b, e_w1, e_b1, e_w2, e_b2):
    B, T, _ = x.shape
    N = B * T
    NT = T // BT
    cap = int(CAPF * (N / E))
    nslot = E * cap
    tbl = nslot + BTF
    hidden = D * MULT
    cht = N // NW

    qwp = _swap_halves(qw)
    kwp = _swap_halves(kw)
    qbp = _swap_halves(qb)
    kbp = _swap_halves(kb)
    wcat = jnp.concatenate([qw, qwp, kw, kwp, vw, gw], axis=1)
    bcat = jnp.concatenate([qb, qbp, kb, kbp, vb, gb]).reshape(1, 6 * D)

    half = DK // 2
    freqs = 1.0 / (10000.0 ** (jnp.arange(half, dtype=jnp.float32) / half))
    f = jnp.outer(jnp.arange(T, dtype=jnp.float32), freqs)
    cosb = jnp.cos(f)
    sinb = jnp.sin(f)
    cosF = jnp.tile(jnp.concatenate([cosb, cosb], axis=1), (1, H))
    sinF = jnp.tile(jnp.concatenate([-sinb, sinb], axis=1), (1, H))

    rw_pad = jnp.zeros((D, LANES), jnp.float32).at[:, :E].set(router_w)
    rb_pad = jnp.full((1, LANES), -1e30, jnp.float32).at[0, :E].set(router_b)

    body = functools.partial(_attn_router_body, cap=cap, nt=NT)
    x2, xn2, spos1, spos2, g1, g2, stats = pl.pallas_call(
        body,
        grid=(B, NT),
        in_specs=[
            pl.BlockSpec((1, BT, D), lambda b, t: (b, t, 0)),
            pl.BlockSpec((1, D), lambda b, t: (0, 0)),
            pl.BlockSpec((1, D), lambda b, t: (0, 0)),
            pl.BlockSpec((D, 6 * D), lambda b, t: (0, 0)),
            pl.BlockSpec((D, D), lambda b, t: (0, 0)),
            pl.BlockSpec((1, 6 * D), lambda b, t: (0, 0)),
            pl.BlockSpec((1, D), lambda b, t: (0, 0)),
            pl.BlockSpec((BT, D), lambda b, t: (t, 0)),
            pl.BlockSpec((BT, D), lambda b, t: (t, 0)),
            pl.BlockSpec((D, LANES), lambda b, t: (0, 0)),
            pl.BlockSpec((1, LANES), lambda b, t: (0, 0)),
        ],
        out_specs=[
            pl.BlockSpec((1, BT, D), lambda b, t: (b, t, 0)),
            pl.BlockSpec((BT, D), lambda b, t: (b * (T // BT) + t, 0)),
            pl.BlockSpec((BT, 1), lambda b, t: (b * (T // BT) + t, 0)),
            pl.BlockSpec((BT, 1), lambda b, t: (b * (T // BT) + t, 0)),
            pl.BlockSpec((BT, 16), lambda b, t: (b * (T // BT) + t, 0)),
            pl.BlockSpec((BT, 16), lambda b, t: (b * (T // BT) + t, 0)),
            pl.BlockSpec((8, LANES), lambda b, t: (0, 0)),
        ],
        out_shape=[
            jax.ShapeDtypeStruct((B, T, D), jnp.float32),
            jax.ShapeDtypeStruct((N, D), jnp.float32),
            jax.ShapeDtypeStruct((N, 1), jnp.int32),
            jax.ShapeDtypeStruct((N, 1), jnp.int32),
            jax.ShapeDtypeStruct((N, 16), jnp.float32),
            jax.ShapeDtypeStruct((N, 16), jnp.float32),
            jax.ShapeDtypeStruct((8, LANES), jnp.float32),
        ],
        scratch_shapes=[
            pltpu.VMEM((D, D), jnp.float32),
            pltpu.VMEM((8, D), jnp.float32),
            pltpu.VMEM((8, LANES), jnp.float32),
        ],
        compiler_params=pltpu.CompilerParams(
            dimension_semantics=("arbitrary", "arbitrary")),
    )(x, norm1_scale.reshape(1, D), norm2_scale.reshape(1, D),
      wcat, ow, bcat, ob.reshape(1, D), cosF, sinF, rw_pad, rb_pad)

    s1f = spos1.reshape(N)
    s2f = spos2.reshape(N)

    mesh = plsc.VectorSubcoreMesh(core_axis_name="c", subcore_axis_name="s",
                                  num_cores=2, num_subcores=16)

    dispatch = pl.kernel(
        functools.partial(_dispatch_body, cht=cht),
        out_type=jax.ShapeDtypeStruct((tbl, D), jnp.float32),
        mesh=mesh,
        scratch_types=[
            pltpu.VMEM((cht,), jnp.int32),
            pltpu.VMEM((cht,), jnp.int32),
            pltpu.VMEM((cht, D), jnp.float32),
            pltpu.SemaphoreType.DMA,
            pltpu.SemaphoreType.DMA,
        ],
    )
    xe = dispatch(xn2, s1f, s2f)

    nbf = cap // BTF
    nsteps = E * nbf
    ffn = functools.partial(_ffn_body, nsteps=nsteps)
    y = pl.pallas_call(
        ffn,
        grid=(nsteps + 1,),
        in_specs=[
            pl.BlockSpec((BTF, D),
                         lambda s: (jnp.minimum(s, E * (cap // BTF) - 1), 0)),
            pl.BlockSpec((1, D, hidden),
                         lambda s: (jnp.minimum(s // (cap // BTF), E - 1), 0, 0)),
            pl.BlockSpec((1, 1, hidden),
                         lambda s: (jnp.minimum(s // (cap // BTF), E - 1), 0, 0)),
            pl.BlockSpec((1, hidden, D),
                         lambda s: (jnp.minimum(s // (cap // BTF), E - 1), 0, 0)),
            pl.BlockSpec((1, 1, D),
                         lambda s: (jnp.minimum(s // (cap // BTF), E - 1), 0, 0)),
        ],
        out_specs=pl.BlockSpec((BTF, D), lambda s: (s, 0)),
        out_shape=jax.ShapeDtypeStruct((nslot + BTF, D), jnp.float32),
        compiler_params=pltpu.CompilerParams(
            dimension_semantics=("arbitrary",)),
    )(xe, e_w1, e_b1.reshape(E, 1, hidden), e_w2, e_b2.reshape(E, 1, D))

    combine = pl.kernel(
        functools.partial(_combine_body, cht=cht),
        out_type=jax.ShapeDtypeStruct((N, D), jnp.float32),
        mesh=mesh,
        scratch_types=[
            pltpu.VMEM((CHC,), jnp.int32),
            pltpu.VMEM((CHC,), jnp.int32),
            pltpu.VMEM((CHC, 16), jnp.float32),
            pltpu.VMEM((CHC, 16), jnp.float32),
            pltpu.VMEM((CHC, D), jnp.float32),
            pltpu.VMEM((CHC, D), jnp.float32),
            pltpu.VMEM((CHC, D), jnp.float32),
            pltpu.SemaphoreType.DMA,
            pltpu.SemaphoreType.DMA,
        ],
    )
    final = combine(x2.reshape(N, D), y, s1f, s2f, g1, g2)

    imp = stats[2, :E]
    load = stats[1, :E]
    aux = jnp.sum((imp / jnp.sum(imp)) * (load / jnp.sum(load))) * float(E * E)
    return final.reshape(B, T, D), aux
